# Initial kernel scaffold; baseline (speedup 1.0000x reference)
import jax
import jax.numpy as jnp
from jax.experimental import pallas as pl


def kernel(x, edge_index, mask, batch, W0, b0, g0, be0, W1, b1, g1, be1, W2, b2, g2, be2, Wout, bout):
    raise NotImplementedError("write your pallas kernel here")


# speedup vs baseline: 2.5274x; 2.5274x over previous
import functools

import jax
import jax.numpy as jnp
from jax import lax
from jax.experimental import pallas as pl
from jax.experimental.pallas import tpu as pltpu

_N = 10000
_E = 320000
_H = 128
_G = 16


def _tc0_body(x_ref, mask_ref, deg_ref, w_ref, u_ref, dinv_ref):
    deg = deg_ref[...] + 1.0
    dinv = lax.rsqrt(deg)
    h = x_ref[...] * mask_ref[...]
    u = jnp.dot(h, w_ref[...], preferred_element_type=jnp.float32) * dinv
    u_ref[...] = u
    dinv_ref[...] = dinv


def _tcmid_body(acc_ref, u_ref, dinv_ref, mask_ref, b_ref, g_ref, be_ref,
                w_ref, out_ref):
    z = (acc_ref[...] + u_ref[...]) * dinv_ref[...] + b_ref[...]
    mu = jnp.mean(z, axis=0, keepdims=True)
    var = jnp.mean((z - mu) ** 2, axis=0, keepdims=True)
    h = (z - mu) * lax.rsqrt(var + 1e-5) * g_ref[...] + be_ref[...]
    h = jnp.maximum(h, 0.0) * mask_ref[...]
    out_ref[...] = jnp.dot(h, w_ref[...],
                           preferred_element_type=jnp.float32) * dinv_ref[...]


def _tcfin_body(acc_ref, u_ref, dinv_ref, mask_ref, b_ref, g_ref, be_ref,
                batch_ref, wout_ref, bout_ref, out_ref):
    z = (acc_ref[...] + u_ref[...]) * dinv_ref[...] + b_ref[...]
    mu = jnp.mean(z, axis=0, keepdims=True)
    var = jnp.mean((z - mu) ** 2, axis=0, keepdims=True)
    h = (z - mu) * lax.rsqrt(var + 1e-5) * g_ref[...] + be_ref[...]
    h = jnp.maximum(h, 0.0) * mask_ref[...]
    gid = lax.broadcasted_iota(jnp.int32, (_N, _G), 1)
    p = (batch_ref[...] == gid).astype(jnp.float32)
    sums = lax.dot_general(p, h, (((0,), (0,)), ((), ())),
                           preferred_element_type=jnp.float32)
    ones = jnp.ones((_N, 1), jnp.float32)
    counts = lax.dot_general(p, ones, (((0,), (0,)), ((), ())),
                             preferred_element_type=jnp.float32)
    rep = sums / jnp.maximum(counts, 1.0)
    out_ref[...] = jnp.dot(rep, wout_ref[...],
                           preferred_element_type=jnp.float32) + bout_ref[...]


def _tc0(x, mask2, deg, w):
    return pl.pallas_call(
        _tc0_body,
        out_shape=(jax.ShapeDtypeStruct((_N, _H), jnp.float32),
                   jax.ShapeDtypeStruct((_N, 1), jnp.float32)),
    )(x, mask2, deg, w)


def _tcmid(acc, u, dinv, mask2, b, g, be, w):
    return pl.pallas_call(
        _tcmid_body,
        out_shape=jax.ShapeDtypeStruct((_N, _H), jnp.float32),
    )(acc, u, dinv, mask2, b, g, be, w)


def _tcfin(acc, u, dinv, mask2, b, g, be, batch2, wout, bout):
    return pl.pallas_call(
        _tcfin_body,
        out_shape=jax.ShapeDtypeStruct((_G, 1), jnp.float32),
    )(acc, u, dinv, mask2, b, g, be, batch2, wout, bout)


def kernel(x, edge_index, mask, batch, W0, b0, g0, be0, W1, b1, g1, be1,
           W2, b2, g2, be2, Wout, bout):
    src = edge_index[0]
    dst = edge_index[1]
    mask2 = mask[:, None]
    batch2 = batch[:, None]

    deg = jax.ops.segment_sum(jnp.ones((_E,), jnp.float32), dst,
                              num_segments=_N)[:, None]

    def scatter(u):
        return jax.ops.segment_sum(u[src], dst, num_segments=_N)

    b0r = b0[None, :]
    g0r = g0[None, :]
    be0r = be0[None, :]
    b1r = b1[None, :]
    g1r = g1[None, :]
    be1r = be1[None, :]
    b2r = b2[None, :]
    g2r = g2[None, :]
    be2r = be2[None, :]
    boutr = bout[None, :]

    u0, dinv = _tc0(x, mask2, deg, W0)
    acc = scatter(u0)
    u1 = _tcmid(acc, u0, dinv, mask2, b0r, g0r, be0r, W1)
    acc = scatter(u1)
    u2 = _tcmid(acc, u1, dinv, mask2, b1r, g1r, be1r, W2)
    acc = scatter(u2)
    out = _tcfin(acc, u2, dinv, mask2, b2r, g2r, be2r, batch2, Wout, boutr)
    return out[:, 0]


# speedup vs baseline: 9.7517x; 3.8585x over previous
import functools

import jax
import jax.numpy as jnp
from jax import lax
from jax.experimental import pallas as pl
from jax.experimental.pallas import tpu as pltpu
from jax.experimental.pallas import tpu_sc as plsc

_N = 10000
_E = 320000
_H = 128
_G = 16

_NC = 2
_NS = 16
_NW = _NC * _NS
_CHUNK = 80
_EPW = _E // _NW
_NCHUNKS = _EPW // _CHUNK
_NPAD = 10112
_RPW = _NPAD // _NS

_mesh = plsc.VectorSubcoreMesh(core_axis_name="c", subcore_axis_name="s")


@functools.partial(
    pl.kernel,
    mesh=_mesh,
    out_type=jax.ShapeDtypeStruct((_NC * _NPAD, _H), jnp.float32),
    scratch_types=[
        pltpu.VMEM((_CHUNK,), jnp.int32),
        pltpu.VMEM((_CHUNK, _H), jnp.float32),
        pltpu.VMEM_SHARED((_NPAD, _H), jnp.float32),
    ],
)
def _sc_degree(dst_hbm, ones_hbm, zeros_hbm, out_hbm, dstb, onesb, dacc):
    c = lax.axis_index("c")
    s = lax.axis_index("s")
    pltpu.sync_copy(ones_hbm, onesb)
    pltpu.sync_copy(zeros_hbm.at[pl.ds(s * _RPW, _RPW)],
                    dacc.at[pl.ds(s * _RPW, _RPW)])
    plsc.subcore_barrier()
    base = (c * _NS + s) * _EPW

    def body(i, carry):
        off = base + i * _CHUNK
        pltpu.sync_copy(dst_hbm.at[pl.ds(off, _CHUNK)], dstb)
        pltpu.sync_copy(onesb, dacc.at[dstb], add=True)
        return carry

    lax.fori_loop(0, _NCHUNKS, body, 0)
    plsc.subcore_barrier()
    pltpu.sync_copy(dacc.at[pl.ds(s * _RPW, _RPW)],
                    out_hbm.at[pl.ds(c * _NPAD + s * _RPW, _RPW)])


@functools.partial(
    pl.kernel,
    mesh=_mesh,
    out_type=jax.ShapeDtypeStruct((_NC * _NPAD, _H), jnp.float32),
    scratch_types=[
        pltpu.VMEM((_CHUNK,), jnp.int32),
        pltpu.VMEM((_CHUNK,), jnp.int32),
        pltpu.VMEM((_CHUNK, _H), jnp.float32),
        pltpu.VMEM_SHARED((_NPAD, _H), jnp.float32),
        pltpu.SemaphoreType.DMA,
    ],
)
def _sc_scatter(u_hbm, src_hbm, dst_hbm, zeros_hbm, out_hbm,
                srcb, dstb, rows, acc, sem):
    c = lax.axis_index("c")
    s = lax.axis_index("s")
    pltpu.sync_copy(zeros_hbm.at[pl.ds(s * _RPW, _RPW)],
                    acc.at[pl.ds(s * _RPW, _RPW)])
    plsc.subcore_barrier()
    base = (c * _NS + s) * _EPW

    def body(i, carry):
        off = base + i * _CHUNK
        pltpu.sync_copy(src_hbm.at[pl.ds(off, _CHUNK)], srcb)
        pltpu.sync_copy(dst_hbm.at[pl.ds(off, _CHUNK)], dstb)
        pltpu.async_copy(u_hbm.at[srcb], rows, sem).wait()
        pltpu.sync_copy(rows, acc.at[dstb], add=True)
        return carry

    lax.fori_loop(0, _NCHUNKS, body, 0)
    plsc.subcore_barrier()
    pltpu.sync_copy(acc.at[pl.ds(s * _RPW, _RPW)],
                    out_hbm.at[pl.ds(c * _NPAD + s * _RPW, _RPW)])


def _tc0_body(x_ref, mask_ref, deg_ref, w_ref, u_ref, dinv_ref):
    deg = deg_ref[0, 0:_N, 0:1] + deg_ref[1, 0:_N, 0:1] + 1.0
    dinv = lax.rsqrt(deg)
    h = x_ref[...] * mask_ref[...]
    u = jnp.dot(h, w_ref[...], preferred_element_type=jnp.float32) * dinv
    u_ref[...] = u
    dinv_ref[...] = dinv


def _tcz_body(acc_ref, u_ref, dinv_ref, b_ref, z_ref):
    z_ref[...] = ((acc_ref[0, 0:_N] + acc_ref[1, 0:_N] + u_ref[...])
                  * dinv_ref[...] + b_ref[...])


def _tcu_body(z_ref, mu_ref, rs_ref, g_ref, be_ref, mask_ref, dinv_ref,
              w_ref, u_ref):
    h = (z_ref[...] - mu_ref[...]) * rs_ref[...] * g_ref[...] + be_ref[...]
    h = jnp.maximum(h, 0.0) * mask_ref[...]
    u_ref[...] = jnp.dot(h, w_ref[...],
                         preferred_element_type=jnp.float32) * dinv_ref[...]


def _tcfin_body(z_ref, mu_ref, rs_ref, g_ref, be_ref, mask_ref,
                batch_ref, wout_ref, bout_ref, out_ref):
    h = (z_ref[...] - mu_ref[...]) * rs_ref[...] * g_ref[...] + be_ref[...]
    h = jnp.maximum(h, 0.0) * mask_ref[...]
    gid = lax.broadcasted_iota(jnp.int32, (_N, _G), 1)
    p = (batch_ref[...] == gid).astype(jnp.float32)
    sums = lax.dot_general(p, h, (((0,), (0,)), ((), ())),
                           preferred_element_type=jnp.float32,
                           precision=lax.Precision.HIGHEST)
    ones = jnp.ones((_N, 1), jnp.float32)
    counts = lax.dot_general(p, ones, (((0,), (0,)), ((), ())),
                             preferred_element_type=jnp.float32,
                             precision=lax.Precision.HIGHEST)
    rep = sums / jnp.maximum(counts, 1.0)
    out_ref[...] = jnp.dot(rep, wout_ref[...],
                           preferred_element_type=jnp.float32) + bout_ref[...]


def _tc0(x, mask2, deg, w):
    return pl.pallas_call(
        _tc0_body,
        out_shape=(jax.ShapeDtypeStruct((_N, _H), jnp.float32),
                   jax.ShapeDtypeStruct((_N, 1), jnp.float32)),
    )(x, mask2, deg, w)


def _tcz(acc, u, dinv, b):
    return pl.pallas_call(
        _tcz_body,
        out_shape=jax.ShapeDtypeStruct((_N, _H), jnp.float32),
    )(acc, u, dinv, b)


def _tcu(z, mu, rs, g, be, mask2, dinv, w):
    return pl.pallas_call(
        _tcu_body,
        out_shape=jax.ShapeDtypeStruct((_N, _H), jnp.float32),
    )(z, mu, rs, g, be, mask2, dinv, w)


def _tcfin(z, mu, rs, g, be, mask2, batch2, wout, bout):
    return pl.pallas_call(
        _tcfin_body,
        out_shape=jax.ShapeDtypeStruct((_G, 1), jnp.float32),
    )(z, mu, rs, g, be, mask2, batch2, wout, bout)


def kernel(x, edge_index, mask, batch, W0, b0, g0, be0, W1, b1, g1, be1,
           W2, b2, g2, be2, Wout, bout):
    src = edge_index[0]
    dst = edge_index[1]
    mask2 = mask[:, None]
    batch2 = batch[:, None]

    zeros128 = jnp.zeros((_NPAD, _H), jnp.float32)
    ones128 = jnp.ones((_CHUNK, _H), jnp.float32)

    deg = _sc_degree(dst, ones128, zeros128).reshape(_NC, _NPAD, _H)

    def scatter(u):
        return _sc_scatter(u, src, dst, zeros128).reshape(_NC, _NPAD, _H)

    def stats(z):
        mu = jnp.mean(z, axis=0)
        var = jnp.var(z, axis=0)
        rs = lax.rsqrt(var + 1e-5)
        return mu[None, :], rs[None, :]

    u0, dinv = _tc0(x, mask2, deg, W0)
    z0 = _tcz(scatter(u0), u0, dinv, b0[None, :])
    mu0, rs0 = stats(z0)
    u1 = _tcu(z0, mu0, rs0, g0[None, :], be0[None, :], mask2, dinv, W1)
    z1 = _tcz(scatter(u1), u1, dinv, b1[None, :])
    mu1, rs1 = stats(z1)
    u2 = _tcu(z1, mu1, rs1, g1[None, :], be1[None, :], mask2, dinv, W2)
    z2 = _tcz(scatter(u2), u2, dinv, b2[None, :])
    mu2, rs2 = stats(z2)
    out = _tcfin(z2, mu2, rs2, g2[None, :], be2[None, :], mask2, batch2,
                 Wout, bout[None, :])
    return out[:, 0]


# speedup vs baseline: 14.3308x; 1.4696x over previous
import functools

import jax
import jax.numpy as jnp
from jax import lax
from jax.experimental import pallas as pl
from jax.experimental.pallas import tpu as pltpu
from jax.experimental.pallas import tpu_sc as plsc

_N = 10000
_E = 320000
_H = 128
_G = 16

_NC = 2
_NS = 16
_NW = _NC * _NS
_CHUNK = 80
_EPW = _E // _NW
_NCHUNKS = _EPW // _CHUNK
_NPAD = 10112
_RPW = _NPAD // _NS

_mesh = plsc.VectorSubcoreMesh(core_axis_name="c", subcore_axis_name="s")


@functools.partial(
    pl.kernel,
    mesh=_mesh,
    out_type=jax.ShapeDtypeStruct((_NC * _NPAD, _H), jnp.float32),
    scratch_types=[
        pltpu.VMEM((_CHUNK,), jnp.int32),
        pltpu.VMEM((_CHUNK, _H), jnp.float32),
        pltpu.VMEM_SHARED((_NPAD, _H), jnp.float32),
    ],
)
def _sc_degree(dst_hbm, ones_hbm, zeros_hbm, out_hbm, dstb, onesb, dacc):
    c = lax.axis_index("c")
    s = lax.axis_index("s")
    pltpu.sync_copy(ones_hbm, onesb)
    pltpu.sync_copy(zeros_hbm.at[pl.ds(s * _RPW, _RPW)],
                    dacc.at[pl.ds(s * _RPW, _RPW)])
    plsc.subcore_barrier()
    base = (c * _NS + s) * _EPW

    def body(i, carry):
        off = base + i * _CHUNK
        pltpu.sync_copy(dst_hbm.at[pl.ds(off, _CHUNK)], dstb)
        pltpu.sync_copy(onesb, dacc.at[dstb], add=True)
        return carry

    lax.fori_loop(0, _NCHUNKS, body, 0)
    plsc.subcore_barrier()
    pltpu.sync_copy(dacc.at[pl.ds(s * _RPW, _RPW)],
                    out_hbm.at[pl.ds(c * _NPAD + s * _RPW, _RPW)])


@functools.partial(
    pl.kernel,
    mesh=_mesh,
    out_type=jax.ShapeDtypeStruct((_NC * _NPAD, _H), jnp.float32),
    scratch_types=[
        pltpu.VMEM((_CHUNK,), jnp.int32),
        pltpu.VMEM((_CHUNK,), jnp.int32),
        pltpu.VMEM((_CHUNK,), jnp.int32),
        pltpu.VMEM((_CHUNK,), jnp.int32),
        pltpu.VMEM((_CHUNK, _H), jnp.float32),
        pltpu.VMEM((_CHUNK, _H), jnp.float32),
        pltpu.VMEM_SHARED((_NPAD, _H), jnp.float32),
        pltpu.SemaphoreType.DMA,
        pltpu.SemaphoreType.DMA,
    ],
)
def _sc_scatter(u_hbm, src_hbm, dst_hbm, zeros_hbm, out_hbm,
                src_a, dst_a, src_b, dst_b, rows_a, rows_b, acc,
                sem_a, sem_b):
    c = lax.axis_index("c")
    s = lax.axis_index("s")
    pltpu.sync_copy(zeros_hbm.at[pl.ds(s * _RPW, _RPW)],
                    acc.at[pl.ds(s * _RPW, _RPW)])
    plsc.subcore_barrier()
    base = (c * _NS + s) * _EPW

    pltpu.sync_copy(src_hbm.at[pl.ds(base, _CHUNK)], src_a)
    pltpu.sync_copy(dst_hbm.at[pl.ds(base, _CHUNK)], dst_a)
    pltpu.async_copy(u_hbm.at[src_a], rows_a, sem_a)

    def pair(j, carry):
        off_b = base + (2 * j + 1) * _CHUNK
        pltpu.sync_copy(src_hbm.at[pl.ds(off_b, _CHUNK)], src_b)
        pltpu.sync_copy(dst_hbm.at[pl.ds(off_b, _CHUNK)], dst_b)
        pltpu.async_copy(u_hbm.at[src_b], rows_b, sem_b)
        pltpu.make_async_copy(u_hbm.at[src_a], rows_a, sem_a).wait()
        pltpu.sync_copy(rows_a, acc.at[dst_a], add=True)

        @pl.when(2 * j + 2 < _NCHUNKS)
        def _():
            off_a = base + (2 * j + 2) * _CHUNK
            pltpu.sync_copy(src_hbm.at[pl.ds(off_a, _CHUNK)], src_a)
            pltpu.sync_copy(dst_hbm.at[pl.ds(off_a, _CHUNK)], dst_a)
            pltpu.async_copy(u_hbm.at[src_a], rows_a, sem_a)

        pltpu.make_async_copy(u_hbm.at[src_b], rows_b, sem_b).wait()
        pltpu.sync_copy(rows_b, acc.at[dst_b], add=True)
        return carry

    lax.fori_loop(0, _NCHUNKS // 2, pair, 0)
    pltpu.make_async_copy(u_hbm.at[src_a], rows_a, sem_a).wait()
    pltpu.sync_copy(rows_a, acc.at[dst_a], add=True)
    plsc.subcore_barrier()
    pltpu.sync_copy(acc.at[pl.ds(s * _RPW, _RPW)],
                    out_hbm.at[pl.ds(c * _NPAD + s * _RPW, _RPW)])


def _tc0_body(x_ref, mask_ref, deg_ref, w_ref, u_ref, dinv_ref):
    deg = deg_ref[0, 0:_N, 0:1] + deg_ref[1, 0:_N, 0:1] + 1.0
    dinv = lax.rsqrt(deg)
    h = x_ref[...] * mask_ref[...]
    u = jnp.dot(h, w_ref[...], preferred_element_type=jnp.float32) * dinv
    u_ref[...] = u
    dinv_ref[...] = dinv


def _tcz_body(acc_ref, u_ref, dinv_ref, b_ref, z_ref):
    z_ref[...] = ((acc_ref[0, 0:_N] + acc_ref[1, 0:_N] + u_ref[...])
                  * dinv_ref[...] + b_ref[...])


def _tcu_body(z_ref, mu_ref, rs_ref, g_ref, be_ref, mask_ref, dinv_ref,
              w_ref, u_ref):
    h = (z_ref[...] - mu_ref[...]) * rs_ref[...] * g_ref[...] + be_ref[...]
    h = jnp.maximum(h, 0.0) * mask_ref[...]
    u_ref[...] = jnp.dot(h, w_ref[...],
                         preferred_element_type=jnp.float32) * dinv_ref[...]


def _tcfin_body(z_ref, mu_ref, rs_ref, g_ref, be_ref, mask_ref,
                batch_ref, wout_ref, bout_ref, out_ref):
    h = (z_ref[...] - mu_ref[...]) * rs_ref[...] * g_ref[...] + be_ref[...]
    h = jnp.maximum(h, 0.0) * mask_ref[...]
    gid = lax.broadcasted_iota(jnp.int32, (_N, _G), 1)
    p = (batch_ref[...] == gid).astype(jnp.float32)
    sums = lax.dot_general(p, h, (((0,), (0,)), ((), ())),
                           preferred_element_type=jnp.float32,
                           precision=lax.Precision.HIGHEST)
    ones = jnp.ones((_N, 1), jnp.float32)
    counts = lax.dot_general(p, ones, (((0,), (0,)), ((), ())),
                             preferred_element_type=jnp.float32,
                             precision=lax.Precision.HIGHEST)
    rep = sums / jnp.maximum(counts, 1.0)
    out_ref[...] = jnp.dot(rep, wout_ref[...],
                           preferred_element_type=jnp.float32) + bout_ref[...]


def _tc0(x, mask2, deg, w):
    return pl.pallas_call(
        _tc0_body,
        out_shape=(jax.ShapeDtypeStruct((_N, _H), jnp.float32),
                   jax.ShapeDtypeStruct((_N, 1), jnp.float32)),
    )(x, mask2, deg, w)


def _tcz(acc, u, dinv, b):
    return pl.pallas_call(
        _tcz_body,
        out_shape=jax.ShapeDtypeStruct((_N, _H), jnp.float32),
    )(acc, u, dinv, b)


def _tcu(z, mu, rs, g, be, mask2, dinv, w):
    return pl.pallas_call(
        _tcu_body,
        out_shape=jax.ShapeDtypeStruct((_N, _H), jnp.float32),
    )(z, mu, rs, g, be, mask2, dinv, w)


def _tcfin(z, mu, rs, g, be, mask2, batch2, wout, bout):
    return pl.pallas_call(
        _tcfin_body,
        out_shape=jax.ShapeDtypeStruct((_G, 1), jnp.float32),
    )(z, mu, rs, g, be, mask2, batch2, wout, bout)


def kernel(x, edge_index, mask, batch, W0, b0, g0, be0, W1, b1, g1, be1,
           W2, b2, g2, be2, Wout, bout):
    src = edge_index[0]
    dst = edge_index[1]
    mask2 = mask[:, None]
    batch2 = batch[:, None]

    zeros128 = jnp.zeros((_NPAD, _H), jnp.float32)
    ones128 = jnp.ones((_CHUNK, _H), jnp.float32)

    deg = _sc_degree(dst, ones128, zeros128).reshape(_NC, _NPAD, _H)

    def scatter(u):
        return _sc_scatter(u, src, dst, zeros128).reshape(_NC, _NPAD, _H)

    def stats(z):
        mu = jnp.mean(z, axis=0)
        var = jnp.var(z, axis=0)
        rs = lax.rsqrt(var + 1e-5)
        return mu[None, :], rs[None, :]

    u0, dinv = _tc0(x, mask2, deg, W0)
    z0 = _tcz(scatter(u0), u0, dinv, b0[None, :])
    mu0, rs0 = stats(z0)
    u1 = _tcu(z0, mu0, rs0, g0[None, :], be0[None, :], mask2, dinv, W1)
    z1 = _tcz(scatter(u1), u1, dinv, b1[None, :])
    mu1, rs1 = stats(z1)
    u2 = _tcu(z1, mu1, rs1, g1[None, :], be1[None, :], mask2, dinv, W2)
    z2 = _tcz(scatter(u2), u2, dinv, b2[None, :])
    mu2, rs2 = stats(z2)
    out = _tcfin(z2, mu2, rs2, g2[None, :], be2[None, :], mask2, batch2,
                 Wout, bout[None, :])
    return out[:, 0]


# speedup vs baseline: 15.2156x; 1.0617x over previous
import functools

import jax
import jax.numpy as jnp
from jax import lax
from jax.experimental import pallas as pl
from jax.experimental.pallas import tpu as pltpu
from jax.experimental.pallas import tpu_sc as plsc

_N = 10000
_E = 320000
_H = 128
_G = 16

_NC = 2
_NS = 16
_NW = _NC * _NS
_CHUNK = 80
_EPW = _E // _NW
_NCHUNKS = _EPW // _CHUNK
_NPAD = 10112
_RPW = _NPAD // _NS

_mesh = plsc.VectorSubcoreMesh(core_axis_name="c", subcore_axis_name="s")


@functools.partial(
    pl.kernel,
    mesh=_mesh,
    out_type=jax.ShapeDtypeStruct((_NC * _NPAD, _H), jnp.float32),
    scratch_types=[
        pltpu.VMEM((_CHUNK,), jnp.int32),
        pltpu.VMEM((_CHUNK,), jnp.int32),
        pltpu.VMEM((_CHUNK, _H), jnp.float32),
        pltpu.VMEM_SHARED((_NPAD, _H), jnp.float32),
        pltpu.SemaphoreType.DMA,
        pltpu.SemaphoreType.DMA,
    ],
)
def _sc_degree(dst_hbm, ones_hbm, zeros_hbm, out_hbm, dst_a, dst_b, onesb,
               dacc, sem_a, sem_b):
    c = lax.axis_index("c")
    s = lax.axis_index("s")
    pltpu.sync_copy(ones_hbm, onesb)
    pltpu.sync_copy(zeros_hbm.at[pl.ds(s * _RPW, _RPW)],
                    dacc.at[pl.ds(s * _RPW, _RPW)])
    plsc.subcore_barrier()
    base = (c * _NS + s) * _EPW

    pltpu.async_copy(dst_hbm.at[pl.ds(base, _CHUNK)], dst_a, sem_a)

    def pair(j, carry):
        off_a = base + (2 * j) * _CHUNK
        off_b = base + (2 * j + 1) * _CHUNK
        pltpu.async_copy(dst_hbm.at[pl.ds(off_b, _CHUNK)], dst_b, sem_b)
        pltpu.make_async_copy(dst_hbm.at[pl.ds(off_a, _CHUNK)], dst_a,
                              sem_a).wait()
        pltpu.sync_copy(onesb, dacc.at[dst_a], add=True)

        @pl.when(2 * j + 2 < _NCHUNKS)
        def _():
            off_a2 = base + (2 * j + 2) * _CHUNK
            pltpu.async_copy(dst_hbm.at[pl.ds(off_a2, _CHUNK)], dst_a, sem_a)

        pltpu.make_async_copy(dst_hbm.at[pl.ds(off_b, _CHUNK)], dst_b,
                              sem_b).wait()
        pltpu.sync_copy(onesb, dacc.at[dst_b], add=True)
        return carry

    lax.fori_loop(0, _NCHUNKS // 2, pair, 0)
    pltpu.make_async_copy(dst_hbm.at[pl.ds(base + (_NCHUNKS - 1) * _CHUNK,
                                           _CHUNK)], dst_a, sem_a).wait()
    pltpu.sync_copy(onesb, dacc.at[dst_a], add=True)
    plsc.subcore_barrier()
    pltpu.sync_copy(dacc.at[pl.ds(s * _RPW, _RPW)],
                    out_hbm.at[pl.ds(c * _NPAD + s * _RPW, _RPW)])


@functools.partial(
    pl.kernel,
    mesh=_mesh,
    out_type=jax.ShapeDtypeStruct((_NC * _NPAD, _H), jnp.float32),
    scratch_types=[
        pltpu.VMEM((_CHUNK,), jnp.int32),
        pltpu.VMEM((_CHUNK,), jnp.int32),
        pltpu.VMEM((_CHUNK,), jnp.int32),
        pltpu.VMEM((_CHUNK,), jnp.int32),
        pltpu.VMEM((_CHUNK, _H), jnp.float32),
        pltpu.VMEM((_CHUNK, _H), jnp.float32),
        pltpu.VMEM_SHARED((_NPAD, _H), jnp.float32),
        pltpu.SemaphoreType.DMA,
        pltpu.SemaphoreType.DMA,
    ],
)
def _sc_scatter(u_hbm, src_hbm, dst_hbm, zeros_hbm, out_hbm,
                src_a, dst_a, src_b, dst_b, rows_a, rows_b, acc,
                sem_a, sem_b):
    c = lax.axis_index("c")
    s = lax.axis_index("s")
    pltpu.sync_copy(zeros_hbm.at[pl.ds(s * _RPW, _RPW)],
                    acc.at[pl.ds(s * _RPW, _RPW)])
    plsc.subcore_barrier()
    base = (c * _NS + s) * _EPW

    pltpu.sync_copy(src_hbm.at[pl.ds(base, _CHUNK)], src_a)
    pltpu.sync_copy(dst_hbm.at[pl.ds(base, _CHUNK)], dst_a)
    pltpu.async_copy(u_hbm.at[src_a], rows_a, sem_a)

    def pair(j, carry):
        off_b = base + (2 * j + 1) * _CHUNK
        pltpu.sync_copy(src_hbm.at[pl.ds(off_b, _CHUNK)], src_b)
        pltpu.sync_copy(dst_hbm.at[pl.ds(off_b, _CHUNK)], dst_b)
        pltpu.async_copy(u_hbm.at[src_b], rows_b, sem_b)
        pltpu.make_async_copy(u_hbm.at[src_a], rows_a, sem_a).wait()
        pltpu.sync_copy(rows_a, acc.at[dst_a], add=True)

        @pl.when(2 * j + 2 < _NCHUNKS)
        def _():
            off_a = base + (2 * j + 2) * _CHUNK
            pltpu.sync_copy(src_hbm.at[pl.ds(off_a, _CHUNK)], src_a)
            pltpu.sync_copy(dst_hbm.at[pl.ds(off_a, _CHUNK)], dst_a)
            pltpu.async_copy(u_hbm.at[src_a], rows_a, sem_a)

        pltpu.make_async_copy(u_hbm.at[src_b], rows_b, sem_b).wait()
        pltpu.sync_copy(rows_b, acc.at[dst_b], add=True)
        return carry

    lax.fori_loop(0, _NCHUNKS // 2, pair, 0)
    pltpu.make_async_copy(u_hbm.at[src_a], rows_a, sem_a).wait()
    pltpu.sync_copy(rows_a, acc.at[dst_a], add=True)
    plsc.subcore_barrier()
    pltpu.sync_copy(acc.at[pl.ds(s * _RPW, _RPW)],
                    out_hbm.at[pl.ds(c * _NPAD + s * _RPW, _RPW)])


def _tc0_body(x_ref, mask_ref, deg_ref, w_ref, u_ref, dinv_ref):
    deg = deg_ref[0, 0:_N, 0:1] + deg_ref[1, 0:_N, 0:1] + 1.0
    dinv = lax.rsqrt(deg)
    h = x_ref[...] * mask_ref[...]
    u = jnp.dot(h, w_ref[...], preferred_element_type=jnp.float32) * dinv
    u_ref[...] = u
    dinv_ref[...] = dinv


def _tcz_body(acc_ref, u_ref, dinv_ref, b_ref, z_ref):
    z_ref[...] = ((acc_ref[0, 0:_N] + acc_ref[1, 0:_N] + u_ref[...])
                  * dinv_ref[...] + b_ref[...])


def _tcu_body(z_ref, mu_ref, rs_ref, g_ref, be_ref, mask_ref, dinv_ref,
              w_ref, u_ref):
    h = (z_ref[...] - mu_ref[...]) * rs_ref[...] * g_ref[...] + be_ref[...]
    h = jnp.maximum(h, 0.0) * mask_ref[...]
    u_ref[...] = jnp.dot(h, w_ref[...],
                         preferred_element_type=jnp.float32) * dinv_ref[...]


def _tcfin_body(z_ref, mu_ref, rs_ref, g_ref, be_ref, mask_ref,
                batch_ref, wout_ref, bout_ref, out_ref):
    h = (z_ref[...] - mu_ref[...]) * rs_ref[...] * g_ref[...] + be_ref[...]
    h = jnp.maximum(h, 0.0) * mask_ref[...]
    gid = lax.broadcasted_iota(jnp.int32, (_N, _G), 1)
    p = (batch_ref[...] == gid).astype(jnp.float32)
    sums = lax.dot_general(p, h, (((0,), (0,)), ((), ())),
                           preferred_element_type=jnp.float32,
                           precision=lax.Precision.HIGHEST)
    ones = jnp.ones((_N, 1), jnp.float32)
    counts = lax.dot_general(p, ones, (((0,), (0,)), ((), ())),
                             preferred_element_type=jnp.float32,
                             precision=lax.Precision.HIGHEST)
    rep = sums / jnp.maximum(counts, 1.0)
    out_ref[...] = jnp.dot(rep, wout_ref[...],
                           preferred_element_type=jnp.float32) + bout_ref[...]


def _tc0(x, mask2, deg, w):
    return pl.pallas_call(
        _tc0_body,
        out_shape=(jax.ShapeDtypeStruct((_N, _H), jnp.float32),
                   jax.ShapeDtypeStruct((_N, 1), jnp.float32)),
    )(x, mask2, deg, w)


def _tcz(acc, u, dinv, b):
    return pl.pallas_call(
        _tcz_body,
        out_shape=jax.ShapeDtypeStruct((_N, _H), jnp.float32),
    )(acc, u, dinv, b)


def _tcu(z, mu, rs, g, be, mask2, dinv, w):
    return pl.pallas_call(
        _tcu_body,
        out_shape=jax.ShapeDtypeStruct((_N, _H), jnp.float32),
    )(z, mu, rs, g, be, mask2, dinv, w)


def _tcfin(z, mu, rs, g, be, mask2, batch2, wout, bout):
    return pl.pallas_call(
        _tcfin_body,
        out_shape=jax.ShapeDtypeStruct((_G, 1), jnp.float32),
    )(z, mu, rs, g, be, mask2, batch2, wout, bout)


def kernel(x, edge_index, mask, batch, W0, b0, g0, be0, W1, b1, g1, be1,
           W2, b2, g2, be2, Wout, bout):
    src = edge_index[0]
    dst = edge_index[1]
    mask2 = mask[:, None]
    batch2 = batch[:, None]

    zeros128 = jnp.zeros((_NPAD, _H), jnp.float32)
    ones128 = jnp.ones((_CHUNK, _H), jnp.float32)

    deg = _sc_degree(dst, ones128, zeros128).reshape(_NC, _NPAD, _H)

    def scatter(u):
        return _sc_scatter(u, src, dst, zeros128).reshape(_NC, _NPAD, _H)

    def stats(z):
        mu = jnp.mean(z, axis=0)
        var = jnp.var(z, axis=0)
        rs = lax.rsqrt(var + 1e-5)
        return mu[None, :], rs[None, :]

    u0, dinv = _tc0(x, mask2, deg, W0)
    z0 = _tcz(scatter(u0), u0, dinv, b0[None, :])
    mu0, rs0 = stats(z0)
    u1 = _tcu(z0, mu0, rs0, g0[None, :], be0[None, :], mask2, dinv, W1)
    z1 = _tcz(scatter(u1), u1, dinv, b1[None, :])
    mu1, rs1 = stats(z1)
    u2 = _tcu(z1, mu1, rs1, g1[None, :], be1[None, :], mask2, dinv, W2)
    z2 = _tcz(scatter(u2), u2, dinv, b2[None, :])
    mu2, rs2 = stats(z2)
    out = _tcfin(z2, mu2, rs2, g2[None, :], be2[None, :], mask2, batch2,
                 Wout, bout[None, :])
    return out[:, 0]
